# baseline (speedup 1.0000x reference)
import functools

import jax
import jax.numpy as jnp
from jax import lax
from jax.experimental import pallas as pl
from jax.experimental.pallas import tpu as pltpu
from jax.experimental.pallas import tpu_sc as plsc

_NC = 2
_NS = 16
_NW = _NC * _NS
_B = 80


def _make_sc_aggregate(n_nodes, n_edges, d):
  e_per_tile = n_edges // _NW
  n_batches = e_per_tile // _B
  assert n_batches * _B == e_per_tile
  n_pad = ((n_nodes + _NS * _B - 1) // (_NS * _B)) * (_NS * _B)
  rows_per_tile = n_pad // _NS
  n_slices = d // 16

  mesh = plsc.VectorSubcoreMesh(
      core_axis_name="c", subcore_axis_name="s",
      num_cores=_NC, num_subcores=_NS)

  @functools.partial(
      pl.kernel,
      out_type=jax.ShapeDtypeStruct((_NC, n_pad, d), jnp.float32),
      mesh=mesh,
      scratch_types=[
          pltpu.VMEM_SHARED((n_pad, d), jnp.float32),
          pltpu.VMEM((_B, d), jnp.float32),
          pltpu.VMEM((_B, d), jnp.float32),
          pltpu.VMEM((n_batches, _B), jnp.int32),
          pltpu.VMEM((n_batches, _B), jnp.float32),
          pltpu.VMEM((_B,), jnp.int32),
          pltpu.VMEM((_B,), jnp.int32),
          pltpu.VMEM((_B,), jnp.int32),
          pltpu.VMEM((_B,), jnp.int32),
          pltpu.SemaphoreType.DMA,
          pltpu.SemaphoreType.DMA,
          pltpu.SemaphoreType.DMA,
          pltpu.SemaphoreType.DMA,
          pltpu.SemaphoreType.DMA,
      ],
      compiler_params=pltpu.CompilerParams(use_tc_tiling_on_sc=False),
  )
  def agg(h_hbm, sd_hbm, w_hbm, out_hbm,
          acc, rows_a, rows_b, sd_v, w_v,
          src_a, dst_a, src_b, dst_b,
          sem_a, sem_b, sem_sa, sem_sb, sem_z):
    c = lax.axis_index("c")
    s = lax.axis_index("s")
    t = c * _NS + s

    stage_sd = pltpu.async_copy(sd_hbm.at[t], sd_v, sem_sa)
    stage_w = pltpu.async_copy(w_hbm.at[t], w_v, sem_sb)
    zero = jnp.zeros((16,), jnp.float32)
    for j in range(_B):
      for k in range(n_slices):
        rows_a[j, pl.ds(k * 16, 16)] = zero
        rows_b[j, pl.ds(k * 16, 16)] = zero
    row0 = s * rows_per_tile
    for off in range(0, rows_per_tile, _B):
      pltpu.async_copy(rows_a, acc.at[pl.ds(row0 + off, _B)], sem_z)
    for off in range(0, rows_per_tile, _B):
      pltpu.make_async_copy(rows_a, acc.at[pl.ds(row0 + off, _B)], sem_z).wait()
    stage_sd.wait()
    stage_w.wait()
    plsc.subcore_barrier()

    def unpack(b, src_i, dst_i):
      for g in range(_B // 16):
        v = sd_v[b, pl.ds(g * 16, 16)]
        src_i[pl.ds(g * 16, 16)] = jnp.bitwise_and(v, 0xFFFF)
        dst_i[pl.ds(g * 16, 16)] = lax.shift_right_logical(v, 16)

    def scale(rows_v, b):
      for g in range(_B // 16):
        w16 = w_v[b, pl.ds(g * 16, 16)]
        for lane in range(16):
          j = g * 16 + lane
          wv = jnp.broadcast_to(w16[lane], (16,))
          for k in range(n_slices):
            rows_v[j, pl.ds(k * 16, 16)] = rows_v[j, pl.ds(k * 16, 16)] * wv

    def gather(rows_v, sem, src_i):
      pltpu.async_copy(h_hbm.at[src_i], rows_v, sem)

    def gwait(rows_v, sem, src_i):
      pltpu.make_async_copy(h_hbm.at[src_i], rows_v, sem).wait()

    def scatter(rows_v, sem, dst_i):
      pltpu.async_copy(rows_v, acc.at[dst_i], sem, add=True)

    def swait(rows_v, sem, dst_i):
      pltpu.make_async_copy(rows_v, acc.at[dst_i], sem).wait()

    unpack(0, src_b, dst_b)
    scatter(rows_b, sem_sb, dst_b)
    unpack(0, src_a, dst_a)
    gather(rows_a, sem_a, src_a)

    def body(b2, carry):
      b = 2 * b2
      swait(rows_b, sem_sb, dst_b)
      unpack(b + 1, src_b, dst_b)
      gather(rows_b, sem_b, src_b)
      gwait(rows_a, sem_a, src_a)
      scale(rows_a, b)
      scatter(rows_a, sem_sa, dst_a)
      gwait(rows_b, sem_b, src_b)
      scale(rows_b, b + 1)
      scatter(rows_b, sem_sb, dst_b)
      swait(rows_a, sem_sa, dst_a)
      unpack(b + 2, src_a, dst_a)
      gather(rows_a, sem_a, src_a)
      return carry

    lax.fori_loop(0, (n_batches - 1) // 2, body, 0)
    last = n_batches - 1
    swait(rows_b, sem_sb, dst_b)
    gwait(rows_a, sem_a, src_a)
    scale(rows_a, last)
    scatter(rows_a, sem_sa, dst_a)
    swait(rows_a, sem_sa, dst_a)
    plsc.subcore_barrier()

    pltpu.sync_copy(acc.at[pl.ds(row0, rows_per_tile)],
                    out_hbm.at[c, pl.ds(row0, rows_per_tile)])

  return agg


def _tc_matmul(x, w):
  m, k = x.shape
  n = w.shape[1]
  bm = 1000

  def body(x_ref, w_ref, o_ref):
    o_ref[...] = jnp.dot(x_ref[...], w_ref[...],
                         preferred_element_type=jnp.float32)

  return pl.pallas_call(
      body,
      grid=(m // bm,),
      in_specs=[pl.BlockSpec((bm, k), lambda i: (i, 0)),
                pl.BlockSpec((k, n), lambda i: (0, 0))],
      out_specs=pl.BlockSpec((bm, n), lambda i: (i, 0)),
      out_shape=jax.ShapeDtypeStruct((m, n), jnp.float32),
  )(x, w)


def _tc_add_relu_matmul(parts, w):
  _, m, k = parts.shape
  n = w.shape[1]
  bm = 1024

  def body(p_ref, w_ref, o_ref):
    h1 = jnp.maximum(p_ref[0] + p_ref[1], 0.0)
    o_ref[...] = jnp.dot(h1, w_ref[...], preferred_element_type=jnp.float32)

  return pl.pallas_call(
      body,
      grid=(m // bm,),
      in_specs=[pl.BlockSpec((2, bm, k), lambda i: (0, i, 0)),
                pl.BlockSpec((k, n), lambda i: (0, 0))],
      out_specs=pl.BlockSpec((bm, n), lambda i: (i, 0)),
      out_shape=jax.ShapeDtypeStruct((m, n), jnp.float32),
  )(parts, w)


def _tc_add_softmax(parts):
  _, m, n = parts.shape
  bm = 1024

  def body(p_ref, o_ref):
    z = p_ref[0] + p_ref[1]
    z = z - jnp.max(z, axis=-1, keepdims=True)
    e = jnp.exp(z)
    o_ref[...] = e / jnp.sum(e, axis=-1, keepdims=True)

  return pl.pallas_call(
      body,
      grid=(m // bm,),
      in_specs=[pl.BlockSpec((2, bm, n), lambda i: (0, i, 0))],
      out_specs=pl.BlockSpec((bm, n), lambda i: (i, 0)),
      out_shape=jax.ShapeDtypeStruct((m, n), jnp.float32),
  )(parts)


def kernel(x, edge_index, edge_weight, W1, W2):
  n = x.shape[0]
  e = edge_index.shape[1]
  nb = e // (_NW * _B)
  src = edge_index[0].astype(jnp.int32)
  dst = edge_index[1].astype(jnp.int32)
  sd = ((dst << 16) | src).reshape(_NW, nb, _B)
  w = edge_weight.astype(jnp.float32).reshape(_NW, nb, _B)

  h = _tc_matmul(x, W1)
  agg1 = _make_sc_aggregate(n, e, W1.shape[1])(h, sd, w)
  h2 = _tc_add_relu_matmul(agg1, W2)
  agg2 = _make_sc_aggregate(n, e, W2.shape[1])(h2, sd, w)
  return _tc_add_softmax(agg2)[:n]


# speedup vs baseline: 1.0742x; 1.0742x over previous
import functools

import jax
import jax.numpy as jnp
from jax import lax
from jax.experimental import pallas as pl
from jax.experimental.pallas import tpu as pltpu
from jax.experimental.pallas import tpu_sc as plsc

_NC = 2
_NS = 16
_NW = _NC * _NS
_B = 80


def _make_sc_aggregate(n_nodes, n_edges, d):
  e_per_tile = n_edges // _NW
  n_batches = e_per_tile // _B
  assert n_batches * _B == e_per_tile
  n_pad = ((n_nodes + _NS * _B - 1) // (_NS * _B)) * (_NS * _B)
  rows_per_tile = n_pad // _NS
  n_slices = d // 16

  mesh = plsc.VectorSubcoreMesh(
      core_axis_name="c", subcore_axis_name="s",
      num_cores=_NC, num_subcores=_NS)

  @functools.partial(
      pl.kernel,
      out_type=jax.ShapeDtypeStruct((_NC, n_pad, d), jnp.float32),
      mesh=mesh,
      scratch_types=[
          pltpu.VMEM_SHARED((n_pad, d), jnp.float32),
          pltpu.VMEM((_B, d), jnp.float32),
          pltpu.VMEM((_B, d), jnp.float32),
          pltpu.VMEM((n_batches, _B), jnp.int32),
          pltpu.VMEM((n_batches, _B), jnp.float32),
          pltpu.VMEM((_B,), jnp.int32),
          pltpu.VMEM((_B,), jnp.int32),
          pltpu.VMEM((_B,), jnp.int32),
          pltpu.VMEM((_B,), jnp.int32),
          pltpu.SemaphoreType.DMA,
          pltpu.SemaphoreType.DMA,
          pltpu.SemaphoreType.DMA,
          pltpu.SemaphoreType.DMA,
          pltpu.SemaphoreType.DMA,
      ],
      compiler_params=pltpu.CompilerParams(use_tc_tiling_on_sc=False),
  )
  def agg(h_hbm, sd_hbm, w_hbm, out_hbm,
          acc, rows_a, rows_b, sd_v, w_v,
          src_a, dst_a, src_b, dst_b,
          sem_a, sem_b, sem_sa, sem_sb, sem_z):
    c = lax.axis_index("c")
    s = lax.axis_index("s")
    t = c * _NS + s

    stage_sd = pltpu.async_copy(sd_hbm.at[t], sd_v, sem_sa)
    stage_w = pltpu.async_copy(w_hbm.at[t], w_v, sem_sb)
    zero = jnp.zeros((16,), jnp.float32)
    for j in range(_B):
      for k in range(n_slices):
        rows_a[j, pl.ds(k * 16, 16)] = zero
    row0 = s * rows_per_tile
    for off in range(0, rows_per_tile, _B):
      pltpu.async_copy(rows_a, acc.at[pl.ds(row0 + off, _B)], sem_z)
    for off in range(0, rows_per_tile, _B):
      pltpu.make_async_copy(rows_a, acc.at[pl.ds(row0 + off, _B)], sem_z).wait()
    stage_sd.wait()
    stage_w.wait()
    plsc.subcore_barrier()

    def unpack(b, src_i, dst_i):
      for g in range(_B // 16):
        v = sd_v[b, pl.ds(g * 16, 16)]
        src_i[pl.ds(g * 16, 16)] = jnp.bitwise_and(v, 0xFFFF)
        dst_i[pl.ds(g * 16, 16)] = lax.shift_right_logical(v, 16)

    def scale(rows_v, b):
      for g in range(_B // 16):
        w16 = w_v[b, pl.ds(g * 16, 16)]
        for lane in range(16):
          j = g * 16 + lane
          wv = jnp.broadcast_to(w16[lane], (16,))
          for k in range(n_slices):
            rows_v[j, pl.ds(k * 16, 16)] = rows_v[j, pl.ds(k * 16, 16)] * wv

    def gather(rows_v, sem, src_i):
      pltpu.async_copy(h_hbm.at[src_i], rows_v, sem)

    def gwait(rows_v, sem, src_i):
      pltpu.make_async_copy(h_hbm.at[src_i], rows_v, sem).wait()

    def scatter(rows_v, dst_i):
      pltpu.sync_copy(rows_v, acc.at[dst_i], add=True)

    unpack(0, src_a, dst_a)
    gather(rows_a, sem_a, src_a)

    def body(b2, carry):
      b = 2 * b2
      unpack(b + 1, src_b, dst_b)
      gather(rows_b, sem_b, src_b)
      gwait(rows_a, sem_a, src_a)
      scale(rows_a, b)
      scatter(rows_a, dst_a)
      unpack(b + 2, src_a, dst_a)
      gather(rows_a, sem_a, src_a)
      gwait(rows_b, sem_b, src_b)
      scale(rows_b, b + 1)
      scatter(rows_b, dst_b)
      return carry

    lax.fori_loop(0, (n_batches - 1) // 2, body, 0)
    last = n_batches - 1
    gwait(rows_a, sem_a, src_a)
    scale(rows_a, last)
    scatter(rows_a, dst_a)
    plsc.subcore_barrier()

    pltpu.sync_copy(acc.at[pl.ds(row0, rows_per_tile)],
                    out_hbm.at[c, pl.ds(row0, rows_per_tile)])

  return agg


def _tc_matmul(x, w):
  m, k = x.shape
  n = w.shape[1]
  bm = 1000

  def body(x_ref, w_ref, o_ref):
    o_ref[...] = jnp.dot(x_ref[...], w_ref[...],
                         preferred_element_type=jnp.float32)

  return pl.pallas_call(
      body,
      grid=(m // bm,),
      in_specs=[pl.BlockSpec((bm, k), lambda i: (i, 0)),
                pl.BlockSpec((k, n), lambda i: (0, 0))],
      out_specs=pl.BlockSpec((bm, n), lambda i: (i, 0)),
      out_shape=jax.ShapeDtypeStruct((m, n), jnp.float32),
  )(x, w)


def _tc_add_relu_matmul(parts, w):
  _, m, k = parts.shape
  n = w.shape[1]
  bm = 1024

  def body(p_ref, w_ref, o_ref):
    h1 = jnp.maximum(p_ref[0] + p_ref[1], 0.0)
    o_ref[...] = jnp.dot(h1, w_ref[...], preferred_element_type=jnp.float32)

  return pl.pallas_call(
      body,
      grid=(m // bm,),
      in_specs=[pl.BlockSpec((2, bm, k), lambda i: (0, i, 0)),
                pl.BlockSpec((k, n), lambda i: (0, 0))],
      out_specs=pl.BlockSpec((bm, n), lambda i: (i, 0)),
      out_shape=jax.ShapeDtypeStruct((m, n), jnp.float32),
  )(parts, w)


def _tc_add_softmax(parts):
  _, m, n = parts.shape
  bm = 1024

  def body(p_ref, o_ref):
    z = p_ref[0] + p_ref[1]
    z = z - jnp.max(z, axis=-1, keepdims=True)
    e = jnp.exp(z)
    o_ref[...] = e / jnp.sum(e, axis=-1, keepdims=True)

  return pl.pallas_call(
      body,
      grid=(m // bm,),
      in_specs=[pl.BlockSpec((2, bm, n), lambda i: (0, i, 0))],
      out_specs=pl.BlockSpec((bm, n), lambda i: (i, 0)),
      out_shape=jax.ShapeDtypeStruct((m, n), jnp.float32),
  )(parts)


def kernel(x, edge_index, edge_weight, W1, W2):
  n = x.shape[0]
  e = edge_index.shape[1]
  nb = e // (_NW * _B)
  src = edge_index[0].astype(jnp.int32)
  dst = edge_index[1].astype(jnp.int32)
  sd = ((dst << 16) | src).reshape(_NW, nb, _B)
  w = edge_weight.astype(jnp.float32).reshape(_NW, nb, _B)

  h = _tc_matmul(x, W1)
  agg1 = _make_sc_aggregate(n, e, W1.shape[1])(h, sd, w)
  h2 = _tc_add_relu_matmul(agg1, W2)
  agg2 = _make_sc_aggregate(n, e, W2.shape[1])(h2, sd, w)
  return _tc_add_softmax(agg2)[:n]


# speedup vs baseline: 1.1382x; 1.0596x over previous
import functools

import jax
import jax.numpy as jnp
from jax import lax
from jax.experimental import pallas as pl
from jax.experimental.pallas import tpu as pltpu
from jax.experimental.pallas import tpu_sc as plsc

_NC = 2
_NS = 16
_NW = _NC * _NS
_B = 80


def _make_sc_aggregate(n_nodes, n_edges, d):
  e_per_tile = n_edges // _NW
  n_batches = e_per_tile // _B
  assert n_batches * _B == e_per_tile
  n_pad = ((n_nodes + _NS * _B - 1) // (_NS * _B)) * (_NS * _B)
  rows_per_tile = n_pad // _NS
  n_slices = d // 16

  mesh = plsc.VectorSubcoreMesh(
      core_axis_name="c", subcore_axis_name="s",
      num_cores=_NC, num_subcores=_NS)

  stage_table = 2 * n_pad * d * 4 <= 6 * 1024 * 1024

  @functools.partial(
      pl.kernel,
      out_type=jax.ShapeDtypeStruct((_NC, n_pad, d), jnp.float32),
      mesh=mesh,
      scratch_types=([pltpu.VMEM_SHARED((n_pad, d), jnp.float32)]
                     if stage_table else []) + [
          pltpu.VMEM_SHARED((n_pad, d), jnp.float32),
          pltpu.VMEM((_B, d), jnp.float32),
          pltpu.VMEM((_B, d), jnp.float32),
          pltpu.VMEM((n_batches, _B), jnp.int32),
          pltpu.VMEM((n_batches, _B), jnp.float32),
          pltpu.VMEM((_B,), jnp.int32),
          pltpu.VMEM((_B,), jnp.int32),
          pltpu.VMEM((_B,), jnp.int32),
          pltpu.VMEM((_B,), jnp.int32),
          pltpu.SemaphoreType.DMA,
          pltpu.SemaphoreType.DMA,
          pltpu.SemaphoreType.DMA,
          pltpu.SemaphoreType.DMA,
          pltpu.SemaphoreType.DMA,
      ],
      compiler_params=pltpu.CompilerParams(use_tc_tiling_on_sc=False),
  )
  def agg(h_hbm, sd_hbm, w_hbm, out_hbm, *scratch):
    if stage_table:
      h_s, acc, rows_a, rows_b, sd_v, w_v = scratch[:6]
    else:
      acc, rows_a, rows_b, sd_v, w_v = scratch[:5]
    (src_a, dst_a, src_b, dst_b,
     sem_a, sem_b, sem_sa, sem_sb, sem_z) = scratch[len(scratch) - 9:]
    c = lax.axis_index("c")
    s = lax.axis_index("s")
    t = c * _NS + s
    row0 = s * rows_per_tile

    stage_sd = pltpu.async_copy(sd_hbm.at[t], sd_v, sem_sa)
    stage_w = pltpu.async_copy(w_hbm.at[t], w_v, sem_sb)
    if stage_table:
      tab_copy = pltpu.async_copy(h_hbm.at[pl.ds(row0, rows_per_tile)],
                                  h_s.at[pl.ds(row0, rows_per_tile)], sem_z)
    zero = jnp.zeros((16,), jnp.float32)
    for j in range(_B):
      for k in range(n_slices):
        rows_a[j, pl.ds(k * 16, 16)] = zero
    if stage_table:
      tab_copy.wait()
    for off in range(0, rows_per_tile, _B):
      pltpu.async_copy(rows_a, acc.at[pl.ds(row0 + off, _B)], sem_z)
    for off in range(0, rows_per_tile, _B):
      pltpu.make_async_copy(rows_a, acc.at[pl.ds(row0 + off, _B)], sem_z).wait()
    stage_sd.wait()
    stage_w.wait()
    plsc.subcore_barrier()
    tab = h_s if stage_table else h_hbm

    def unpack(b, src_i, dst_i):
      for g in range(_B // 16):
        v = sd_v[b, pl.ds(g * 16, 16)]
        src_i[pl.ds(g * 16, 16)] = jnp.bitwise_and(v, 0xFFFF)
        dst_i[pl.ds(g * 16, 16)] = lax.shift_right_logical(v, 16)

    def scale(rows_v, b):
      for g in range(_B // 16):
        w16 = w_v[b, pl.ds(g * 16, 16)]
        for lane in range(16):
          j = g * 16 + lane
          wv = jnp.broadcast_to(w16[lane], (16,))
          for k in range(n_slices):
            rows_v[j, pl.ds(k * 16, 16)] = rows_v[j, pl.ds(k * 16, 16)] * wv

    def gather(rows_v, sem, src_i):
      pltpu.async_copy(tab.at[src_i], rows_v, sem)

    def gwait(rows_v, sem, src_i):
      pltpu.make_async_copy(tab.at[src_i], rows_v, sem).wait()

    def scatter(rows_v, dst_i):
      pltpu.sync_copy(rows_v, acc.at[dst_i], add=True)

    unpack(0, src_a, dst_a)
    gather(rows_a, sem_a, src_a)

    def body(b2, carry):
      b = 2 * b2
      unpack(b + 1, src_b, dst_b)
      gather(rows_b, sem_b, src_b)
      gwait(rows_a, sem_a, src_a)
      scale(rows_a, b)
      scatter(rows_a, dst_a)
      unpack(b + 2, src_a, dst_a)
      gather(rows_a, sem_a, src_a)
      gwait(rows_b, sem_b, src_b)
      scale(rows_b, b + 1)
      scatter(rows_b, dst_b)
      return carry

    lax.fori_loop(0, (n_batches - 1) // 2, body, 0)
    last = n_batches - 1
    gwait(rows_a, sem_a, src_a)
    scale(rows_a, last)
    scatter(rows_a, dst_a)
    plsc.subcore_barrier()

    pltpu.sync_copy(acc.at[pl.ds(row0, rows_per_tile)],
                    out_hbm.at[c, pl.ds(row0, rows_per_tile)])

  return agg


def _tc_matmul(x, w):
  m, k = x.shape
  n = w.shape[1]
  bm = 1000

  def body(x_ref, w_ref, o_ref):
    o_ref[...] = jnp.dot(x_ref[...], w_ref[...],
                         preferred_element_type=jnp.float32)

  return pl.pallas_call(
      body,
      grid=(m // bm,),
      in_specs=[pl.BlockSpec((bm, k), lambda i: (i, 0)),
                pl.BlockSpec((k, n), lambda i: (0, 0))],
      out_specs=pl.BlockSpec((bm, n), lambda i: (i, 0)),
      out_shape=jax.ShapeDtypeStruct((m, n), jnp.float32),
  )(x, w)


def _tc_add_relu_matmul(parts, w):
  _, m, k = parts.shape
  n = w.shape[1]
  bm = 1024

  def body(p_ref, w_ref, o_ref):
    h1 = jnp.maximum(p_ref[0] + p_ref[1], 0.0)
    o_ref[...] = jnp.dot(h1, w_ref[...], preferred_element_type=jnp.float32)

  return pl.pallas_call(
      body,
      grid=(m // bm,),
      in_specs=[pl.BlockSpec((2, bm, k), lambda i: (0, i, 0)),
                pl.BlockSpec((k, n), lambda i: (0, 0))],
      out_specs=pl.BlockSpec((bm, n), lambda i: (i, 0)),
      out_shape=jax.ShapeDtypeStruct((m, n), jnp.float32),
  )(parts, w)


def _tc_add_softmax(parts):
  _, m, n = parts.shape
  bm = 1024

  def body(p_ref, o_ref):
    z = p_ref[0] + p_ref[1]
    z = z - jnp.max(z, axis=-1, keepdims=True)
    e = jnp.exp(z)
    o_ref[...] = e / jnp.sum(e, axis=-1, keepdims=True)

  return pl.pallas_call(
      body,
      grid=(m // bm,),
      in_specs=[pl.BlockSpec((2, bm, n), lambda i: (0, i, 0))],
      out_specs=pl.BlockSpec((bm, n), lambda i: (i, 0)),
      out_shape=jax.ShapeDtypeStruct((m, n), jnp.float32),
  )(parts)


def kernel(x, edge_index, edge_weight, W1, W2):
  n = x.shape[0]
  e = edge_index.shape[1]
  nb = e // (_NW * _B)
  src = edge_index[0].astype(jnp.int32)
  dst = edge_index[1].astype(jnp.int32)
  sd = ((dst << 16) | src).reshape(_NW, nb, _B)
  w = edge_weight.astype(jnp.float32).reshape(_NW, nb, _B)

  h = _tc_matmul(x, W1)
  agg1 = _make_sc_aggregate(n, e, W1.shape[1])(h, sd, w)
  h2 = _tc_add_relu_matmul(agg1, W2)
  agg2 = _make_sc_aggregate(n, e, W2.shape[1])(h2, sd, w)
  return _tc_add_softmax(agg2)[:n]
